# Initial kernel scaffold; baseline (speedup 1.0000x reference)
import jax
import jax.numpy as jnp
from jax.experimental import pallas as pl


def kernel(token_ids, W):
    raise NotImplementedError("write your pallas kernel here")


# speedup vs baseline: 4.0881x; 4.0881x over previous
import functools

import jax
import jax.numpy as jnp
from jax import lax
from jax.experimental import pallas as pl
from jax.experimental.pallas import tpu as pltpu
from jax.experimental.pallas import tpu_sc as plsc

NUM_WORKERS = 32
CHUNK = 128


@functools.partial(jax.jit, static_argnums=(2, 3))
def _gather(idx, table, n_per_w, d):
    n = idx.shape[0]
    chunks_per_w = n_per_w // CHUNK

    mesh = plsc.VectorSubcoreMesh(core_axis_name="c", subcore_axis_name="s")

    @functools.partial(
        pl.kernel,
        out_type=jax.ShapeDtypeStruct((n, d), jnp.float32),
        mesh=mesh,
        scratch_types=[
            pltpu.VMEM((n_per_w,), jnp.int32),
            pltpu.VMEM((CHUNK, d), jnp.float32),
            pltpu.SemaphoreType.DMA,
        ],
        compiler_params=pltpu.CompilerParams(use_tc_tiling_on_sc=False),
    )
    def k(idx_hbm, table_hbm, out_hbm, idx_v, rows_v, gsem):
        wid = lax.axis_index("s") * 2 + lax.axis_index("c")
        r0 = wid * n_per_w
        pltpu.sync_copy(idx_hbm.at[pl.ds(r0, n_per_w)], idx_v)

        def body(j, carry):
            ix = idx_v.at[pl.ds(j * CHUNK, CHUNK)]
            pltpu.async_copy(table_hbm.at[ix], rows_v, gsem).wait()
            pltpu.sync_copy(rows_v, out_hbm.at[pl.ds(r0 + j * CHUNK, CHUNK)])
            return carry

        lax.fori_loop(0, chunks_per_w, body, 0)

    return k(idx, table)


def kernel(token_ids, W):
    b, l = token_ids.shape
    v, d = W.shape
    n = b * l
    n_per_w = n // NUM_WORKERS
    idx = token_ids.astype(jnp.int32).reshape(n)
    out = _gather(idx, W, n_per_w, d)
    return out.reshape(b, l, d)


# speedup vs baseline: 4.6180x; 1.1296x over previous
import functools

import jax
import jax.numpy as jnp
from jax import lax
from jax.experimental import pallas as pl
from jax.experimental.pallas import tpu as pltpu
from jax.experimental.pallas import tpu_sc as plsc

NUM_WORKERS = 32
CHUNK = 128
K = 5
NBUF = 2


@functools.partial(jax.jit, static_argnums=(2, 3))
def _gather(idx, table, n_per_w, d):
    n = idx.shape[0]
    groups = n_per_w // (K * CHUNK)
    gsz = K * CHUNK

    mesh = plsc.VectorSubcoreMesh(core_axis_name="c", subcore_axis_name="s")

    @functools.partial(
        pl.kernel,
        out_type=jax.ShapeDtypeStruct((n, d), jnp.float32),
        mesh=mesh,
        scratch_types=[
            pltpu.VMEM((n_per_w,), jnp.int32),
            pltpu.VMEM((NBUF, gsz, d), jnp.float32),
            pltpu.SemaphoreType.DMA,
            pltpu.SemaphoreType.DMA,
            pltpu.SemaphoreType.DMA,
        ],
        compiler_params=pltpu.CompilerParams(use_tc_tiling_on_sc=False),
    )
    def k(idx_hbm, table_hbm, out_hbm, idx_v, rows_v, gsem, osem0, osem1):
        wid = lax.axis_index("s") * 2 + lax.axis_index("c")
        r0 = wid * n_per_w
        pltpu.sync_copy(idx_hbm.at[pl.ds(r0, n_per_w)], idx_v)
        osems = (osem0, osem1)

        def out_slice(g):
            return out_hbm.at[pl.ds(r0 + g * gsz, gsz)]

        def group(g, b, wait_prev):
            buf = rows_v.at[b]
            if wait_prev:
                pltpu.make_async_copy(buf, out_slice(g - NBUF), osems[b]).wait()
            handles = [
                pltpu.async_copy(
                    table_hbm.at[idx_v.at[pl.ds((g * K + j) * CHUNK, CHUNK)]],
                    buf.at[pl.ds(j * CHUNK, CHUNK)],
                    gsem,
                )
                for j in range(K)
            ]
            for h in handles:
                h.wait()
            pltpu.async_copy(buf, out_slice(g), osems[b])

        group(0, 0, False)
        group(1, 1, False)

        def body(i, carry):
            group(NBUF * i + 2, 0, True)
            group(NBUF * i + 3, 1, True)
            return carry

        lax.fori_loop(0, (groups - NBUF) // NBUF, body, 0)

        for b in range(NBUF):
            pltpu.make_async_copy(
                rows_v.at[b], out_slice(groups - NBUF + b), osems[b]
            ).wait()

    return k(idx, table)


def kernel(token_ids, W):
    b, l = token_ids.shape
    v, d = W.shape
    n = b * l
    n_per_w = n // NUM_WORKERS
    idx = token_ids.astype(jnp.int32).reshape(n)
    out = _gather(idx, W, n_per_w, d)
    return out.reshape(b, l, d)
